# Initial kernel scaffold; baseline (speedup 1.0000x reference)
import jax
import jax.numpy as jnp
from jax.experimental import pallas as pl


def kernel(x, edge_index, W1, b1, W2, b2, W3, b3):
    raise NotImplementedError("write your pallas kernel here")


# speedup vs baseline: 1.9890x; 1.9890x over previous
import functools
import jax
import jax.numpy as jnp
from jax.experimental import pallas as pl

N = 100000


def _mm_kernel(x_ref, w_ref, b_ref, o_ref):
    o_ref[...] = x_ref[...] @ w_ref[...] + b_ref[...]


def _matmul_bias(x, W, b):
    BN = 2000
    return pl.pallas_call(
        _mm_kernel,
        grid=(x.shape[0] // BN,),
        in_specs=[
            pl.BlockSpec((BN, x.shape[1]), lambda i: (i, 0)),
            pl.BlockSpec((W.shape[0], W.shape[1]), lambda i: (0, 0)),
            pl.BlockSpec((W.shape[1],), lambda i: (0,)),
        ],
        out_specs=pl.BlockSpec((BN, W.shape[1]), lambda i: (i, 0)),
        out_shape=jax.ShapeDtypeStruct((x.shape[0], W.shape[1]), jnp.float32),
    )(x, W, b)


def kernel(x, edge_index, W1, b1, W2, b2, W3, b3):
    x = x.astype(jnp.float32)
    src, dst = edge_index[0], edge_index[1]
    deg = jnp.zeros((N,), jnp.float32).at[dst].add(1.0) + 1.0
    dis = deg ** -0.5
    d = dis[:, None]
    g1 = x * d
    s1 = jnp.zeros_like(g1).at[dst].add(g1[src])
    h1 = jax.nn.relu(_matmul_bias(d * (s1 + g1), W1, b1))
    g2 = _matmul_bias(h1, W2, jnp.zeros_like(b2)) * d
    s2 = jnp.zeros_like(g2).at[dst].add(g2[src])
    h2 = jax.nn.relu(d * (s2 + g2) + b2)
    return _matmul_bias(h2, W3, b3)


# speedup vs baseline: 31.3234x; 15.7480x over previous
import functools

import jax
import jax.numpy as jnp
from jax import lax
from jax.experimental import pallas as pl
from jax.experimental.pallas import tpu as pltpu
from jax.experimental.pallas import tpu_sc as plsc

N = 100000
NP = 100096
E = 3200000
EPAD = 3276800
ER = EPAD // 128
RPT = NP // 16
DRB = 782
BN = 4352
GRID = NP // BN

_mesh = plsc.VectorSubcoreMesh(core_axis_name="c", subcore_axis_name="s")
_sc_params = pltpu.CompilerParams(use_tc_tiling_on_sc=False)


@functools.partial(
    pl.kernel,
    out_type=jax.ShapeDtypeStruct((2, NP), jnp.float32),
    mesh=_mesh,
    compiler_params=_sc_params,
    scratch_types=[
        pltpu.VMEM((128,), jnp.float32),
        pltpu.VMEM((RPT,), jnp.float32),
        pltpu.VMEM((16, 128), jnp.int32),
        pltpu.VMEM_SHARED((NP,), jnp.float32),
    ],
)
def _deg_sc(dst_hbm, ones_hbm, zeros_hbm, out_hbm, ones_v, zbuf, dstv, acc):
    c = lax.axis_index("c")
    s = lax.axis_index("s")
    pltpu.sync_copy(ones_hbm, ones_v)
    pltpu.sync_copy(zeros_hbm, zbuf)
    pltpu.sync_copy(zbuf, acc.at[pl.ds(s * RPT, RPT)])
    plsc.subcore_barrier()
    base = (s * 2 + c) * 800

    def chunk(i, _):
        off = base + i * 16
        pltpu.sync_copy(dst_hbm.at[pl.ds(off, 16), :], dstv)
        for j in range(16):
            pltpu.sync_copy(ones_v, acc.at[dstv.at[j]], add=True)
        return 0

    lax.fori_loop(0, 50, chunk, 0)
    plsc.subcore_barrier()
    pltpu.sync_copy(acc.at[pl.ds(s * RPT, RPT)], zbuf)
    pltpu.sync_copy(zbuf, out_hbm.at[c, pl.ds(s * RPT, RPT)])


def _make_row_agg(n_table_rows, c_stride, s_stride, n_chunks):
    @functools.partial(
        pl.kernel,
        out_type=jax.ShapeDtypeStruct((2, NP, 16), jnp.float32),
        mesh=_mesh,
        compiler_params=_sc_params,
        scratch_types=[
            pltpu.VMEM((DRB, 16), jnp.float32),
            pltpu.VMEM((8, 128), jnp.int32),
            pltpu.VMEM((8, 128), jnp.int32),
            pltpu.VMEM((8, 128, 16), jnp.float32),
            pltpu.SemaphoreType.DMA((8,)),
            pltpu.VMEM_SHARED((NP, 16), jnp.float32),
        ],
    )
    def agg(table_hbm, src_hbm, dst_hbm, zeros_hbm, out_hbm,
            zbuf, srcv, dstv, rows, sems, acc):
        c = lax.axis_index("c")
        s = lax.axis_index("s")
        pltpu.sync_copy(zeros_hbm, zbuf)
        for k in range(RPT // DRB):
            pltpu.sync_copy(zbuf, acc.at[pl.ds(s * RPT + k * DRB, DRB), :])
        plsc.subcore_barrier()
        base = c * c_stride + s * s_stride

        def chunk(i, _):
            off = base + i * 8
            pltpu.sync_copy(src_hbm.at[pl.ds(off, 8), :], srcv)
            pltpu.sync_copy(dst_hbm.at[pl.ds(off, 8), :], dstv)
            descs = [
                pltpu.async_copy(table_hbm.at[srcv.at[j]], rows.at[j], sems.at[j])
                for j in range(8)
            ]
            for j in range(8):
                descs[j].wait()
                pltpu.sync_copy(rows.at[j], acc.at[dstv.at[j]], add=True)
            return 0

        lax.fori_loop(0, n_chunks, chunk, 0)
        plsc.subcore_barrier()
        for k in range(RPT // DRB):
            r0 = s * RPT + k * DRB
            pltpu.sync_copy(acc.at[pl.ds(r0, DRB), :], zbuf)
            pltpu.sync_copy(zbuf, out_hbm.at[c, pl.ds(r0, DRB), :])

    return agg


_agg_l1 = _make_row_agg(NP, 800, 1600, 100)
_agg_l2 = _make_row_agg(2 * NP, ER, 1600, 200)


def _t0_body(deg_ref, x_ref, g1_ref):
    dis = lax.rsqrt(deg_ref[0] + deg_ref[1] + 1.0)
    g1_ref[...] = x_ref[...] * dis[:, None]


def _t0(deg, x_p):
    return pl.pallas_call(
        _t0_body,
        grid=(GRID,),
        in_specs=[
            pl.BlockSpec((2, BN), lambda i: (0, i)),
            pl.BlockSpec((BN, 16), lambda i: (i, 0)),
        ],
        out_specs=pl.BlockSpec((BN, 16), lambda i: (i, 0)),
        out_shape=jax.ShapeDtypeStruct((NP, 16), jnp.float32),
    )(deg, x_p)


def _t1_body(deg_ref, s1_ref, g1_ref, w1_ref, b1_ref, w2_ref, g2_ref):
    dis = lax.rsqrt(deg_ref[0] + deg_ref[1] + 1.0)[:, None]
    y1 = (dis * (s1_ref[0] + s1_ref[1] + g1_ref[...])) @ w1_ref[...] + b1_ref[...]
    t = jax.nn.relu(y1) @ w2_ref[...]
    g2 = t * dis
    g2_ref[0] = g2[:, :16]
    g2_ref[1] = g2[:, 16:]


def _t1(deg, s1, g1, W1, b1, W2):
    return pl.pallas_call(
        _t1_body,
        grid=(GRID,),
        in_specs=[
            pl.BlockSpec((2, BN), lambda i: (0, i)),
            pl.BlockSpec((2, BN, 16), lambda i: (0, i, 0)),
            pl.BlockSpec((BN, 16), lambda i: (i, 0)),
            pl.BlockSpec((16, 48), lambda i: (0, 0)),
            pl.BlockSpec((48,), lambda i: (0,)),
            pl.BlockSpec((48, 32), lambda i: (0, 0)),
        ],
        out_specs=pl.BlockSpec((2, BN, 16), lambda i: (0, i, 0)),
        out_shape=jax.ShapeDtypeStruct((2, NP, 16), jnp.float32),
    )(deg, s1, g1, W1, b1, W2)


def _t2_body(deg_ref, s2_ref, g2_ref, b2_ref, w3_ref, b3_ref, o_ref):
    dis = lax.rsqrt(deg_ref[0] + deg_ref[1] + 1.0)[:, None]
    ya = dis * (s2_ref[0] + g2_ref[0])
    yb = dis * (s2_ref[1] + g2_ref[1])
    h2 = jax.nn.relu(jnp.concatenate([ya, yb], axis=1) + b2_ref[...])
    o_ref[...] = h2 @ w3_ref[...] + b3_ref[...]


def _t2(deg, s2, g2, b2, W3, b3):
    return pl.pallas_call(
        _t2_body,
        grid=(GRID,),
        in_specs=[
            pl.BlockSpec((2, BN), lambda i: (0, i)),
            pl.BlockSpec((2, BN, 16), lambda i: (0, i, 0)),
            pl.BlockSpec((2, BN, 16), lambda i: (0, i, 0)),
            pl.BlockSpec((32,), lambda i: (0,)),
            pl.BlockSpec((32, 10), lambda i: (0, 0)),
            pl.BlockSpec((10,), lambda i: (0,)),
        ],
        out_specs=pl.BlockSpec((BN, 10), lambda i: (i, 0)),
        out_shape=jax.ShapeDtypeStruct((NP, 10), jnp.float32),
    )(deg, s2, g2, b2, W3, b3)


def kernel(x, edge_index, W1, b1, W2, b2, W3, b3):
    x = x.astype(jnp.float32)
    ei = edge_index.astype(jnp.int32)
    pad = jnp.full((EPAD - E,), N, jnp.int32)
    srcr = jnp.concatenate([ei[0], pad]).reshape(ER, 128)
    dstr = jnp.concatenate([ei[1], pad]).reshape(ER, 128)
    x_p = jnp.zeros((NP, 16), jnp.float32).at[:N].set(x)

    ones128 = jnp.ones((128,), jnp.float32)
    zeros1 = jnp.zeros((RPT,), jnp.float32)
    zeros16 = jnp.zeros((DRB, 16), jnp.float32)

    deg = _deg_sc(dstr, ones128, zeros1)
    g1 = _t0(deg, x_p)
    s1 = _agg_l1(g1, srcr, dstr, zeros16)
    g2 = _t1(deg, s1, g1, W1, b1, W2)

    src2 = jnp.concatenate([srcr, srcr + NP], axis=0)
    dst2 = jnp.concatenate([dstr, dstr], axis=0)
    s2 = _agg_l2(g2.reshape(2 * NP, 16), src2, dst2, zeros16)

    out = _t2(deg, s2, g2, b2, W3, b3)
    return out[:N]


# speedup vs baseline: 33.3026x; 1.0632x over previous
import functools

import jax
import jax.numpy as jnp
from jax import lax
from jax.experimental import pallas as pl
from jax.experimental.pallas import tpu as pltpu
from jax.experimental.pallas import tpu_sc as plsc

N = 100000
NP = 100096
E = 3200000
EPAD = 3276800
ER = EPAD // 128
RPT = NP // 16
DRB = 782
BN = 4352
GRID = NP // BN

_mesh = plsc.VectorSubcoreMesh(core_axis_name="c", subcore_axis_name="s")
_sc_params = pltpu.CompilerParams(use_tc_tiling_on_sc=False)


@functools.partial(
    pl.kernel,
    out_type=jax.ShapeDtypeStruct((2, NP), jnp.float32),
    mesh=_mesh,
    compiler_params=_sc_params,
    scratch_types=[
        pltpu.VMEM((128,), jnp.float32),
        pltpu.VMEM((RPT,), jnp.float32),
        pltpu.VMEM((16, 128), jnp.int32),
        pltpu.VMEM_SHARED((NP,), jnp.float32),
    ],
)
def _deg_sc(dst_hbm, ones_hbm, zeros_hbm, out_hbm, ones_v, zbuf, dstv, acc):
    c = lax.axis_index("c")
    s = lax.axis_index("s")
    pltpu.sync_copy(ones_hbm, ones_v)
    pltpu.sync_copy(zeros_hbm, zbuf)
    pltpu.sync_copy(zbuf, acc.at[pl.ds(s * RPT, RPT)])
    plsc.subcore_barrier()
    base = (s * 2 + c) * 800

    def chunk(i, _):
        off = base + i * 16
        pltpu.sync_copy(dst_hbm.at[pl.ds(off, 16), :], dstv)
        for j in range(16):
            pltpu.sync_copy(ones_v, acc.at[dstv.at[j]], add=True)
        return 0

    lax.fori_loop(0, 50, chunk, 0)
    plsc.subcore_barrier()
    pltpu.sync_copy(acc.at[pl.ds(s * RPT, RPT)], zbuf)
    pltpu.sync_copy(zbuf, out_hbm.at[c, pl.ds(s * RPT, RPT)])


def _make_row_agg(n_tables, c_stride, s_stride, n_chunks):
    @functools.partial(
        pl.kernel,
        out_type=jax.ShapeDtypeStruct((2, NP, 16), jnp.float32),
        mesh=_mesh,
        compiler_params=_sc_params,
        scratch_types=[
            pltpu.VMEM((DRB, 16), jnp.float32),
            pltpu.VMEM((8, 128), jnp.int32),
            pltpu.VMEM((8, 128), jnp.int32),
            pltpu.VMEM((8, 128, 16), jnp.float32),
            pltpu.SemaphoreType.DMA((8,)),
            pltpu.SemaphoreType.DMA((8,)),
            pltpu.VMEM_SHARED((NP, 16), jnp.float32),
        ],
    )
    def agg(table_hbm, src_hbm, dst_hbm, zeros_hbm, out_hbm,
            zbuf, srcv, dstv, rows, gsems, ssems, acc):
        c = lax.axis_index("c")
        s = lax.axis_index("s")
        table = table_hbm.at[c * (n_tables - 1)]
        pltpu.sync_copy(zeros_hbm, zbuf)
        for k in range(RPT // DRB):
            pltpu.sync_copy(zbuf, acc.at[pl.ds(s * RPT + k * DRB, DRB), :])
        plsc.subcore_barrier()
        base = c * c_stride + s * s_stride

        def chunk(i, _):
            off = base + i * 8
            pltpu.sync_copy(src_hbm.at[pl.ds(off, 8), :], srcv)
            pltpu.sync_copy(dst_hbm.at[pl.ds(off, 8), :], dstv)
            gd = [
                pltpu.async_copy(table.at[srcv.at[j]], rows.at[j], gsems.at[j])
                for j in range(8)
            ]
            sd = []
            for j in range(8):
                gd[j].wait()
                sd.append(pltpu.async_copy(
                    rows.at[j], acc.at[dstv.at[j]], ssems.at[j], add=True))
            for j in range(8):
                sd[j].wait()
            return 0

        lax.fori_loop(0, n_chunks, chunk, 0)
        plsc.subcore_barrier()
        for k in range(RPT // DRB):
            r0 = s * RPT + k * DRB
            pltpu.sync_copy(acc.at[pl.ds(r0, DRB), :], zbuf)
            pltpu.sync_copy(zbuf, out_hbm.at[c, pl.ds(r0, DRB), :])

    return agg


_agg_l1 = _make_row_agg(1, 800, 1600, 100)
_agg_l2 = _make_row_agg(2, 0, 1600, 200)


def _t0_body(deg_ref, x_ref, g1_ref):
    dis = lax.rsqrt(deg_ref[0] + deg_ref[1] + 1.0)
    g1_ref[...] = x_ref[...] * dis[:, None]


def _t0(deg, x_p):
    return pl.pallas_call(
        _t0_body,
        grid=(GRID,),
        in_specs=[
            pl.BlockSpec((2, BN), lambda i: (0, i)),
            pl.BlockSpec((BN, 16), lambda i: (i, 0)),
        ],
        out_specs=pl.BlockSpec((BN, 16), lambda i: (i, 0)),
        out_shape=jax.ShapeDtypeStruct((NP, 16), jnp.float32),
    )(deg, x_p)


def _t1_body(deg_ref, s1_ref, g1_ref, w1_ref, b1_ref, w2_ref, g2_ref):
    dis = lax.rsqrt(deg_ref[0] + deg_ref[1] + 1.0)[:, None]
    y1 = (dis * (s1_ref[0] + s1_ref[1] + g1_ref[...])) @ w1_ref[...] + b1_ref[...]
    t = jax.nn.relu(y1) @ w2_ref[...]
    g2 = t * dis
    g2_ref[0] = g2[:, :16]
    g2_ref[1] = g2[:, 16:]


def _t1(deg, s1, g1, W1, b1, W2):
    return pl.pallas_call(
        _t1_body,
        grid=(GRID,),
        in_specs=[
            pl.BlockSpec((2, BN), lambda i: (0, i)),
            pl.BlockSpec((2, BN, 16), lambda i: (0, i, 0)),
            pl.BlockSpec((BN, 16), lambda i: (i, 0)),
            pl.BlockSpec((16, 48), lambda i: (0, 0)),
            pl.BlockSpec((48,), lambda i: (0,)),
            pl.BlockSpec((48, 32), lambda i: (0, 0)),
        ],
        out_specs=pl.BlockSpec((2, BN, 16), lambda i: (0, i, 0)),
        out_shape=jax.ShapeDtypeStruct((2, NP, 16), jnp.float32),
    )(deg, s1, g1, W1, b1, W2)


def _t2_body(deg_ref, s2_ref, g2_ref, b2_ref, w3_ref, b3_ref, o_ref):
    dis = lax.rsqrt(deg_ref[0] + deg_ref[1] + 1.0)[:, None]
    ya = dis * (s2_ref[0] + g2_ref[0])
    yb = dis * (s2_ref[1] + g2_ref[1])
    h2 = jax.nn.relu(jnp.concatenate([ya, yb], axis=1) + b2_ref[...])
    o_ref[...] = h2 @ w3_ref[...] + b3_ref[...]


def _t2(deg, s2, g2, b2, W3, b3):
    return pl.pallas_call(
        _t2_body,
        grid=(GRID,),
        in_specs=[
            pl.BlockSpec((2, BN), lambda i: (0, i)),
            pl.BlockSpec((2, BN, 16), lambda i: (0, i, 0)),
            pl.BlockSpec((2, BN, 16), lambda i: (0, i, 0)),
            pl.BlockSpec((32,), lambda i: (0,)),
            pl.BlockSpec((32, 10), lambda i: (0, 0)),
            pl.BlockSpec((10,), lambda i: (0,)),
        ],
        out_specs=pl.BlockSpec((BN, 10), lambda i: (i, 0)),
        out_shape=jax.ShapeDtypeStruct((NP, 10), jnp.float32),
    )(deg, s2, g2, b2, W3, b3)


def kernel(x, edge_index, W1, b1, W2, b2, W3, b3):
    x = x.astype(jnp.float32)
    ei = edge_index.astype(jnp.int32)
    pad = jnp.full((EPAD - E,), N, jnp.int32)
    srcr = jnp.concatenate([ei[0], pad]).reshape(ER, 128)
    dstr = jnp.concatenate([ei[1], pad]).reshape(ER, 128)
    x_p = jnp.zeros((NP, 16), jnp.float32).at[:N].set(x)

    ones128 = jnp.ones((128,), jnp.float32)
    zeros1 = jnp.zeros((RPT,), jnp.float32)
    zeros16 = jnp.zeros((DRB, 16), jnp.float32)

    deg = _deg_sc(dstr, ones128, zeros1)
    g1 = _t0(deg, x_p)
    s1 = _agg_l1(g1.reshape(1, NP, 16), srcr, dstr, zeros16)
    g2 = _t1(deg, s1, g1, W1, b1, W2)
    s2 = _agg_l2(g2, srcr, dstr, zeros16)
    out = _t2(deg, s2, g2, b2, W3, b3)
    return out[:N]
